# Initial kernel scaffold; baseline (speedup 1.0000x reference)
import functools

import jax
import jax.numpy as jnp
from jax import lax
from jax.experimental import pallas as pl
from jax.experimental.pallas import tpu as pltpu
from jax.experimental.pallas import tpu_sc as plsc

N_NODES = 10000
N_EDGES = 320000
D_NODE = 142
D_HID = 128
NUM_GRAPHS = 25
NUM_ENC = 10

SC_CORES = 2
SC_TILES = 16
N_WORKERS = SC_CORES * SC_TILES
EDGES_PER_WORKER = N_EDGES // N_WORKERS
CHUNK = 80
N_CHUNKS = EDGES_PER_WORKER // CHUNK
ROWS_PER_TILE = N_NODES // SC_TILES


def _node_proj_body(a_ref, w_ref, o_ref):
    o_ref[...] = jnp.dot(a_ref[...], w_ref[...],
                         preferred_element_type=jnp.float32)


def _node_proj(atom, w_top):
    blk = 2000
    return pl.pallas_call(
        _node_proj_body,
        grid=(N_NODES // blk,),
        in_specs=[
            pl.BlockSpec((blk, D_NODE), lambda i: (i, 0)),
            pl.BlockSpec((D_NODE, D_HID), lambda i: (0, 0)),
        ],
        out_specs=pl.BlockSpec((blk, D_HID), lambda i: (i, 0)),
        out_shape=jax.ShapeDtypeStruct((N_NODES, D_HID), jnp.float32),
    )(atom, w_top)


def _edge_q_body(bond_ref, dist_ref, inv_ref, w_ref, b_ref, o_ref):
    x = dist_ref[...] * inv_ref[...]
    ef = jnp.concatenate([bond_ref[...], jnp.sin(x), jnp.cos(x)], axis=-1)
    o_ref[...] = jnp.dot(ef, w_ref[...],
                         preferred_element_type=jnp.float32) + b_ref[...]


def _edge_q(bond, dist2d, inv_scales, w_bot, b_msg2d):
    blk = 3200
    d_in = w_bot.shape[0]
    return pl.pallas_call(
        _edge_q_body,
        grid=(N_EDGES // blk,),
        in_specs=[
            pl.BlockSpec((blk, bond.shape[1]), lambda i: (i, 0)),
            pl.BlockSpec((blk, 1), lambda i: (i, 0)),
            pl.BlockSpec((1, NUM_ENC), lambda i: (0, 0)),
            pl.BlockSpec((d_in, D_HID), lambda i: (0, 0)),
            pl.BlockSpec((1, D_HID), lambda i: (0, 0)),
        ],
        out_specs=pl.BlockSpec((blk, D_HID), lambda i: (i, 0)),
        out_shape=jax.ShapeDtypeStruct((N_EDGES, D_HID), jnp.float32),
    )(bond, dist2d, inv_scales, w_bot, b_msg2d)


def _sc_aggregate(p, q, src, dst):
    mesh = plsc.VectorSubcoreMesh(core_axis_name="c", subcore_axis_name="s")

    @functools.partial(
        pl.kernel,
        out_type=jax.ShapeDtypeStruct((SC_CORES, N_NODES, D_HID), jnp.float32),
        mesh=mesh,
        scratch_types=[
            pltpu.VMEM((CHUNK,), jnp.int32),
            pltpu.VMEM((CHUNK,), jnp.int32),
            pltpu.VMEM((CHUNK, D_HID), jnp.float32),
            pltpu.VMEM((CHUNK, D_HID), jnp.float32),
            pltpu.VMEM_SHARED((N_NODES, D_HID), jnp.float32),
            pltpu.SemaphoreType.DMA,
        ],
    )
    def sc_kernel(p_hbm, q_hbm, src_hbm, dst_hbm, out_hbm,
                  src_v, dst_v, g_v, q_v, agg_sh, sem):
        cid = lax.axis_index("c")
        sid = lax.axis_index("s")
        wid = cid * SC_TILES + sid

        @pl.loop(0, CHUNK)
        def _(r):
            for j in range(D_HID // 16):
                g_v.at[pl.ds(r, 1), pl.ds(j * 16, 16)][...] = (
                    jnp.zeros((1, 16), jnp.float32))

        row0 = sid * ROWS_PER_TILE

        @pl.loop(0, ROWS_PER_TILE // CHUNK)
        def _(k):
            pltpu.sync_copy(g_v, agg_sh.at[pl.ds(row0 + k * CHUNK, CHUNK)])

        rem = ROWS_PER_TILE % CHUNK
        if rem:
            pltpu.sync_copy(
                g_v.at[pl.ds(0, rem)],
                agg_sh.at[pl.ds(row0 + (ROWS_PER_TILE // CHUNK) * CHUNK, rem)])

        plsc.subcore_barrier()

        base_w = wid * EDGES_PER_WORKER

        @pl.loop(0, N_CHUNKS)
        def _(t):
            base = base_w + t * CHUNK
            pltpu.sync_copy(src_hbm.at[pl.ds(base, CHUNK)], src_v)
            pltpu.sync_copy(dst_hbm.at[pl.ds(base, CHUNK)], dst_v)
            pltpu.async_copy(q_hbm.at[pl.ds(base, CHUNK)], q_v, sem).wait()
            pltpu.sync_copy(p_hbm.at[src_v], g_v)

            @pl.loop(0, CHUNK)
            def _(r):
                for j in range(D_HID // 16):
                    slc = (pl.ds(r, 1), pl.ds(j * 16, 16))
                    g_v.at[slc][...] = jnp.maximum(
                        g_v.at[slc][...] + q_v.at[slc][...], 0.0)

            pltpu.sync_copy(g_v, agg_sh.at[dst_v], add=True)

        plsc.subcore_barrier()

        pltpu.sync_copy(agg_sh.at[pl.ds(row0, ROWS_PER_TILE)],
                        out_hbm.at[cid].at[pl.ds(row0, ROWS_PER_TILE)])

    return sc_kernel(p, q, src, dst)


def _head_body(a_ref, agg_ref, n2g_ref, wt_ref, wb_ref, bn_ref, wo_ref,
               bo_ref, o_ref):
    agg = agg_ref[0] + agg_ref[1]
    h = jnp.dot(a_ref[...], wt_ref[...],
                preferred_element_type=jnp.float32)
    h = h + jnp.dot(agg, wb_ref[...], preferred_element_type=jnp.float32)
    h = jax.nn.relu(h + bn_ref[...])
    gids = lax.broadcasted_iota(jnp.int32, (N_NODES, NUM_GRAPHS), 1)
    m = (n2g_ref[...] == gids).astype(jnp.float32)
    g_sum = lax.dot_general(m, h, (((0,), (0,)), ((), ())),
                            preferred_element_type=jnp.float32)
    ones = jnp.ones((N_NODES, 1), jnp.float32)
    g_cnt = lax.dot_general(m, ones, (((0,), (0,)), ((), ())),
                            preferred_element_type=jnp.float32)
    g_mean = g_sum / jnp.maximum(g_cnt, 1.0)
    h = h + jnp.dot(m, g_mean, preferred_element_type=jnp.float32)
    o_ref[...] = jax.nn.sigmoid(
        jnp.dot(h, wo_ref[...], preferred_element_type=jnp.float32)
        + bo_ref[...])


def _head(atom, agg2, n2g2d, w_top, w_bot, b_node2d, w_out, b_out2d):
    return pl.pallas_call(
        _head_body,
        out_shape=jax.ShapeDtypeStruct((N_NODES, 1), jnp.float32),
    )(atom, agg2, n2g2d, w_top, w_bot, b_node2d, w_out, b_out2d)


def kernel(atom_feature, edge_index, bond_feature, distance, b_factor,
           node2graph, W_msg, b_msg, W_node, b_node, W_out, b_out):
    src = edge_index[0]
    dst = edge_index[1]
    inv_scales = (1.0 / (2.0 ** jnp.arange(NUM_ENC, dtype=jnp.float32))
                  ).reshape(1, NUM_ENC)

    p = _node_proj(atom_feature, W_msg[:D_NODE])
    q = _edge_q(bond_feature, distance.reshape(-1, 1), inv_scales,
                W_msg[D_NODE:], b_msg.reshape(1, -1))
    agg2 = _sc_aggregate(p, q, src, dst)
    out = _head(atom_feature, agg2, node2graph.reshape(-1, 1),
                W_node[:D_NODE], W_node[D_NODE:], b_node.reshape(1, -1),
                W_out, b_out.reshape(1, 1))
    return out.reshape(-1), b_factor


# speedup vs baseline: 1.8331x; 1.8331x over previous
import functools

import jax
import jax.numpy as jnp
from jax import lax
from jax.experimental import pallas as pl
from jax.experimental.pallas import tpu as pltpu
from jax.experimental.pallas import tpu_sc as plsc

N_NODES = 10000
N_EDGES = 320000
D_NODE = 142
D_HID = 128
NUM_GRAPHS = 25
NUM_ENC = 10

SC_CORES = 2
SC_TILES = 16
N_WORKERS = SC_CORES * SC_TILES
EDGES_PER_WORKER = N_EDGES // N_WORKERS
CHUNK = 80
N_CHUNKS = EDGES_PER_WORKER // CHUNK
ROWS_PER_TILE = 624


def _node_proj_body(a_ref, w_ref, o_ref):
    o_ref[...] = jnp.dot(a_ref[...], w_ref[...],
                         preferred_element_type=jnp.float32)


def _node_proj(atom, w_top):
    blk = 2000
    return pl.pallas_call(
        _node_proj_body,
        grid=(N_NODES // blk,),
        in_specs=[
            pl.BlockSpec((blk, D_NODE), lambda i: (i, 0)),
            pl.BlockSpec((D_NODE, D_HID), lambda i: (0, 0)),
        ],
        out_specs=pl.BlockSpec((blk, D_HID), lambda i: (i, 0)),
        out_shape=jax.ShapeDtypeStruct((N_NODES, D_HID), jnp.float32),
    )(atom, w_top)


def _edge_q_body(bond_ref, dist_ref, inv_ref, w_ref, b_ref, o_ref):
    x = dist_ref[...] * inv_ref[...]
    ef = jnp.concatenate([bond_ref[...], jnp.sin(x), jnp.cos(x)], axis=-1)
    o_ref[...] = jnp.dot(ef, w_ref[...],
                         preferred_element_type=jnp.float32) + b_ref[...]


def _edge_q(bond, dist2d, inv_scales, w_bot, b_msg2d):
    blk = 3200
    d_in = w_bot.shape[0]
    return pl.pallas_call(
        _edge_q_body,
        grid=(N_EDGES // blk,),
        in_specs=[
            pl.BlockSpec((blk, bond.shape[1]), lambda i: (i, 0)),
            pl.BlockSpec((blk, 1), lambda i: (i, 0)),
            pl.BlockSpec((1, NUM_ENC), lambda i: (0, 0)),
            pl.BlockSpec((d_in, D_HID), lambda i: (0, 0)),
            pl.BlockSpec((1, D_HID), lambda i: (0, 0)),
        ],
        out_specs=pl.BlockSpec((blk, D_HID), lambda i: (i, 0)),
        out_shape=jax.ShapeDtypeStruct((N_EDGES, D_HID), jnp.float32),
    )(bond, dist2d, inv_scales, w_bot, b_msg2d)


def _sc_aggregate(p, q, src, dst):
    mesh = plsc.VectorSubcoreMesh(core_axis_name="c", subcore_axis_name="s")

    @functools.partial(
        pl.kernel,
        out_type=jax.ShapeDtypeStruct((SC_CORES, N_NODES, D_HID), jnp.float32),
        mesh=mesh,
        scratch_types=[
            pltpu.VMEM((CHUNK,), jnp.int32),
            pltpu.VMEM((CHUNK,), jnp.int32),
            pltpu.VMEM((CHUNK, D_HID), jnp.float32),
            pltpu.VMEM((CHUNK, D_HID), jnp.float32),
            pltpu.VMEM_SHARED((N_NODES, D_HID), jnp.float32),
            pltpu.SemaphoreType.DMA,
        ],
    )
    def sc_kernel(p_hbm, q_hbm, src_hbm, dst_hbm, out_hbm,
                  src_v, dst_v, g_v, q_v, agg_sh, sem):
        cid = lax.axis_index("c")
        sid = lax.axis_index("s")
        wid = cid * SC_TILES + sid

        @pl.loop(0, CHUNK)
        def _(r):
            for j in range(D_HID // 16):
                g_v.at[pl.ds(r, 1), pl.ds(j * 16, 16)][...] = (
                    jnp.zeros((1, 16), jnp.float32))

        row0 = sid * ROWS_PER_TILE

        @pl.loop(0, ROWS_PER_TILE // CHUNK)
        def _(k):
            pltpu.sync_copy(g_v, agg_sh.at[pl.ds(row0 + k * CHUNK, CHUNK)])

        rem = ROWS_PER_TILE % CHUNK
        if rem:
            pltpu.sync_copy(
                g_v.at[pl.ds(0, rem)],
                agg_sh.at[pl.ds(row0 + (ROWS_PER_TILE // CHUNK) * CHUNK, rem)])

        tail0 = SC_TILES * ROWS_PER_TILE
        tail = N_NODES - tail0

        @pl.when(sid == SC_TILES - 1)
        def _():
            pltpu.sync_copy(g_v.at[pl.ds(0, tail)],
                            agg_sh.at[pl.ds(tail0, tail)])

        plsc.subcore_barrier()

        base_w = wid * EDGES_PER_WORKER

        @pl.loop(0, N_CHUNKS)
        def _(t):
            base = base_w + t * CHUNK
            pltpu.sync_copy(src_hbm.at[pl.ds(base, CHUNK)], src_v)
            pltpu.sync_copy(dst_hbm.at[pl.ds(base, CHUNK)], dst_v)
            pltpu.async_copy(q_hbm.at[pl.ds(base, CHUNK)], q_v, sem).wait()
            pltpu.sync_copy(p_hbm.at[src_v], g_v)

            @pl.loop(0, CHUNK)
            def _(r):
                for j in range(D_HID // 16):
                    slc = (pl.ds(r, 1), pl.ds(j * 16, 16))
                    g_v.at[slc][...] = jnp.maximum(
                        g_v.at[slc][...] + q_v.at[slc][...], 0.0)

            pltpu.sync_copy(g_v, agg_sh.at[dst_v], add=True)

        plsc.subcore_barrier()

        pltpu.sync_copy(agg_sh.at[pl.ds(row0, ROWS_PER_TILE)],
                        out_hbm.at[cid].at[pl.ds(row0, ROWS_PER_TILE)])

        @pl.when(sid == SC_TILES - 1)
        def _():
            pltpu.sync_copy(agg_sh.at[pl.ds(tail0, tail)],
                            out_hbm.at[cid].at[pl.ds(tail0, tail)])

    return sc_kernel(p, q, src, dst)


def _head_body(a_ref, agg_ref, n2g_ref, wt_ref, wb_ref, bn_ref, wo_ref,
               bo_ref, o_ref):
    agg = agg_ref[0] + agg_ref[1]
    h = jnp.dot(a_ref[...], wt_ref[...],
                preferred_element_type=jnp.float32)
    h = h + jnp.dot(agg, wb_ref[...], preferred_element_type=jnp.float32)
    h = jax.nn.relu(h + bn_ref[...])
    gids = lax.broadcasted_iota(jnp.int32, (N_NODES, NUM_GRAPHS), 1)
    m = (n2g_ref[...] == gids).astype(jnp.float32)
    g_sum = lax.dot_general(m, h, (((0,), (0,)), ((), ())),
                            preferred_element_type=jnp.float32)
    ones = jnp.ones((N_NODES, 1), jnp.float32)
    g_cnt = lax.dot_general(m, ones, (((0,), (0,)), ((), ())),
                            preferred_element_type=jnp.float32)
    g_mean = g_sum / jnp.maximum(g_cnt, 1.0)
    h = h + jnp.dot(m, g_mean, preferred_element_type=jnp.float32)
    o_ref[...] = jax.nn.sigmoid(
        jnp.dot(h, wo_ref[...], preferred_element_type=jnp.float32)
        + bo_ref[...])


def _head(atom, agg2, n2g2d, w_top, w_bot, b_node2d, w_out, b_out2d):
    return pl.pallas_call(
        _head_body,
        out_shape=jax.ShapeDtypeStruct((N_NODES, 1), jnp.float32),
    )(atom, agg2, n2g2d, w_top, w_bot, b_node2d, w_out, b_out2d)


def kernel(atom_feature, edge_index, bond_feature, distance, b_factor,
           node2graph, W_msg, b_msg, W_node, b_node, W_out, b_out):
    src = edge_index[0]
    dst = edge_index[1]
    inv_scales = (1.0 / (2.0 ** jnp.arange(NUM_ENC, dtype=jnp.float32))
                  ).reshape(1, NUM_ENC)

    p = _node_proj(atom_feature, W_msg[:D_NODE])
    q = _edge_q(bond_feature, distance.reshape(-1, 1), inv_scales,
                W_msg[D_NODE:], b_msg.reshape(1, -1))
    agg2 = _sc_aggregate(p, q, src, dst)
    out = _head(atom_feature, agg2, node2graph.reshape(-1, 1),
                W_node[:D_NODE], W_node[D_NODE:], b_node.reshape(1, -1),
                W_out, b_out.reshape(1, 1))
    return out.reshape(-1), b_factor


# speedup vs baseline: 5.0027x; 2.7290x over previous
import functools

import jax
import jax.numpy as jnp
from jax import lax
from jax.experimental import pallas as pl
from jax.experimental.pallas import tpu as pltpu
from jax.experimental.pallas import tpu_sc as plsc

N_NODES = 10000
N_EDGES = 320000
D_NODE = 142
D_HID = 128
NUM_GRAPHS = 25
NUM_ENC = 10

SC_CORES = 2
SC_TILES = 16
N_WORKERS = SC_CORES * SC_TILES
EDGES_PER_WORKER = N_EDGES // N_WORKERS
CHUNK = 40
N_CHUNKS = EDGES_PER_WORKER // CHUNK
IDXB = 50
N_BATCH = N_CHUNKS // IDXB
ROWS_PER_TILE = 624


def _node_proj_body(a_ref, w_ref, o_ref):
    o_ref[...] = jnp.dot(a_ref[...], w_ref[...],
                         preferred_element_type=jnp.float32)


def _node_proj(atom, w_top):
    blk = 2000
    return pl.pallas_call(
        _node_proj_body,
        grid=(N_NODES // blk,),
        in_specs=[
            pl.BlockSpec((blk, D_NODE), lambda i: (i, 0)),
            pl.BlockSpec((D_NODE, D_HID), lambda i: (0, 0)),
        ],
        out_specs=pl.BlockSpec((blk, D_HID), lambda i: (i, 0)),
        out_shape=jax.ShapeDtypeStruct((N_NODES, D_HID), jnp.float32),
    )(atom, w_top)


_TWO_OVER_PI = 0.6366197723675814
_PIO2_HI = 1.570800781250
_PIO2_LO = -4.454455103380768e-06


def _sincos(x):
    k = jnp.floor(x * _TWO_OVER_PI + 0.5)
    r = (x - k * _PIO2_HI) - k * _PIO2_LO
    ki = k.astype(jnp.int32)
    r2 = r * r
    sin_r = r * (1.0 + r2 * (-1.0 / 6.0 + r2 * (1.0 / 120.0
                                                + r2 * (-1.0 / 5040.0))))
    cos_r = 1.0 + r2 * (-0.5 + r2 * (1.0 / 24.0 + r2 * (-1.0 / 720.0
                                                        + r2 / 40320.0)))
    swap = (ki & 1) != 0
    s1 = jnp.where(swap, cos_r, sin_r)
    c1 = jnp.where(swap, sin_r, cos_r)
    s = jnp.where((ki & 2) != 0, -s1, s1)
    c = jnp.where(((ki + 1) & 2) != 0, -c1, c1)
    return s, c


def _edge_q_body(bond_ref, dist_ref, inv_ref, wb_ref, ws_ref, wc_ref, b_ref,
                 o_ref):
    x = inv_ref[...] * dist_ref[...]
    s, c = _sincos(x)
    acc = jnp.dot(bond_ref[...], wb_ref[...],
                  preferred_element_type=jnp.float32)
    acc = acc + lax.dot_general(s, ws_ref[...], (((0,), (0,)), ((), ())),
                                preferred_element_type=jnp.float32)
    acc = acc + lax.dot_general(c, wc_ref[...], (((0,), (0,)), ((), ())),
                                preferred_element_type=jnp.float32)
    o_ref[...] = acc + b_ref[...]


def _edge_q(bond, dist_row, inv_col, w_bond, w_sin, w_cos, b_msg2d):
    blk = 6400
    return pl.pallas_call(
        _edge_q_body,
        grid=(N_EDGES // blk,),
        in_specs=[
            pl.BlockSpec((blk, bond.shape[1]), lambda i: (i, 0)),
            pl.BlockSpec((1, blk), lambda i: (0, i)),
            pl.BlockSpec((NUM_ENC, 1), lambda i: (0, 0)),
            pl.BlockSpec((bond.shape[1], D_HID), lambda i: (0, 0)),
            pl.BlockSpec((NUM_ENC, D_HID), lambda i: (0, 0)),
            pl.BlockSpec((NUM_ENC, D_HID), lambda i: (0, 0)),
            pl.BlockSpec((1, D_HID), lambda i: (0, 0)),
        ],
        out_specs=pl.BlockSpec((blk, D_HID), lambda i: (i, 0)),
        out_shape=jax.ShapeDtypeStruct((N_EDGES, D_HID), jnp.float32),
    )(bond, dist_row, inv_col, w_bond, w_sin, w_cos, b_msg2d)


def _sc_aggregate(p, q, src2d, dst2d):
    mesh = plsc.VectorSubcoreMesh(core_axis_name="c", subcore_axis_name="s")

    @functools.partial(
        pl.kernel,
        out_type=jax.ShapeDtypeStruct((SC_CORES, N_NODES, D_HID), jnp.float32),
        mesh=mesh,
        scratch_types=[
            pltpu.VMEM((IDXB, CHUNK), jnp.int32),
            pltpu.VMEM((IDXB, CHUNK), jnp.int32),
            pltpu.VMEM((CHUNK, D_HID), jnp.float32),
            pltpu.VMEM((CHUNK, D_HID), jnp.float32),
            pltpu.VMEM((CHUNK, D_HID), jnp.float32),
            pltpu.VMEM((CHUNK, D_HID), jnp.float32),
            pltpu.VMEM_SHARED((N_NODES, D_HID), jnp.float32),
            pltpu.SemaphoreType.DMA,
            pltpu.SemaphoreType.DMA,
            pltpu.SemaphoreType.DMA,
            pltpu.SemaphoreType.DMA,
            pltpu.SemaphoreType.DMA,
        ],
    )
    def sc_kernel(p_hbm, q_hbm, src_hbm, dst_hbm, out_hbm,
                  src_v, dst_v, g_a, g_b, q_a, q_b, agg_sh,
                  sem_ga, sem_gb, sem_qa, sem_qb, sem_i):
        cid = lax.axis_index("c")
        sid = lax.axis_index("s")
        wid = cid * SC_TILES + sid
        base_w = wid * EDGES_PER_WORKER

        pltpu.async_copy(src_hbm.at[wid, 0], src_v, sem_i)
        pltpu.async_copy(dst_hbm.at[wid, 0], dst_v, sem_i)

        @pl.loop(0, CHUNK)
        def _(r):
            for j in range(D_HID // 16):
                g_a.at[pl.ds(r, 1), pl.ds(j * 16, 16)][...] = (
                    jnp.zeros((1, 16), jnp.float32))

        row0 = sid * ROWS_PER_TILE

        @pl.loop(0, ROWS_PER_TILE // CHUNK)
        def _(k):
            pltpu.sync_copy(g_a, agg_sh.at[pl.ds(row0 + k * CHUNK, CHUNK)])

        rem = ROWS_PER_TILE % CHUNK
        if rem:
            pltpu.sync_copy(
                g_a.at[pl.ds(0, rem)],
                agg_sh.at[pl.ds(row0 + (ROWS_PER_TILE // CHUNK) * CHUNK, rem)])

        tail0 = SC_TILES * ROWS_PER_TILE
        tail = N_NODES - tail0

        @pl.when(sid == SC_TILES - 1)
        def _():
            pltpu.sync_copy(g_a.at[pl.ds(0, tail)],
                            agg_sh.at[pl.ds(tail0, tail)])

        pltpu.make_async_copy(src_hbm.at[wid, 0], src_v, sem_i).wait()
        pltpu.make_async_copy(dst_hbm.at[wid, 0], dst_v, sem_i).wait()
        plsc.subcore_barrier()

        def issue(b, t, g_buf, q_buf, sem_g, sem_q):
            pltpu.async_copy(p_hbm.at[src_v.at[t]], g_buf, sem_g)
            pltpu.async_copy(
                q_hbm.at[pl.ds(base_w + (b * IDXB + t) * CHUNK, CHUNK)],
                q_buf, sem_q)

        def wait(b, t, g_buf, q_buf, sem_g, sem_q):
            pltpu.make_async_copy(p_hbm.at[src_v.at[t]], g_buf, sem_g).wait()
            pltpu.make_async_copy(
                q_hbm.at[pl.ds(base_w + (b * IDXB + t) * CHUNK, CHUNK)],
                q_buf, sem_q).wait()

        def process(t, g_buf, q_buf):
            @pl.loop(0, CHUNK)
            def _(r):
                for j in range(D_HID // 16):
                    slc = (pl.ds(r, 1), pl.ds(j * 16, 16))
                    g_buf.at[slc][...] = jnp.maximum(
                        g_buf.at[slc][...] + q_buf.at[slc][...], 0.0)

            pltpu.sync_copy(g_buf, agg_sh.at[dst_v.at[t]], add=True)

        @pl.loop(0, N_BATCH)
        def _(b):
            issue(b, 0, g_a, q_a, sem_ga, sem_qa)

            @pl.loop(0, IDXB // 2 - 1)
            def _(t2):
                ta = 2 * t2
                issue(b, ta + 1, g_b, q_b, sem_gb, sem_qb)
                wait(b, ta, g_a, q_a, sem_ga, sem_qa)
                process(ta, g_a, q_a)
                issue(b, ta + 2, g_a, q_a, sem_ga, sem_qa)
                wait(b, ta + 1, g_b, q_b, sem_gb, sem_qb)
                process(ta + 1, g_b, q_b)

            issue(b, IDXB - 1, g_b, q_b, sem_gb, sem_qb)
            wait(b, IDXB - 2, g_a, q_a, sem_ga, sem_qa)
            process(IDXB - 2, g_a, q_a)
            wait(b, IDXB - 1, g_b, q_b, sem_gb, sem_qb)
            process(IDXB - 1, g_b, q_b)

            @pl.when(b + 1 < N_BATCH)
            def _():
                pltpu.sync_copy(src_hbm.at[wid, b + 1], src_v)
                pltpu.sync_copy(dst_hbm.at[wid, b + 1], dst_v)

        plsc.subcore_barrier()

        pltpu.sync_copy(agg_sh.at[pl.ds(row0, ROWS_PER_TILE)],
                        out_hbm.at[cid].at[pl.ds(row0, ROWS_PER_TILE)])

        @pl.when(sid == SC_TILES - 1)
        def _():
            pltpu.sync_copy(agg_sh.at[pl.ds(tail0, tail)],
                            out_hbm.at[cid].at[pl.ds(tail0, tail)])

    return sc_kernel(p, q, src2d, dst2d)


def _head_body(a_ref, agg_ref, n2g_ref, wt_ref, wb_ref, bn_ref, wo_ref,
               bo_ref, o_ref):
    agg = agg_ref[0] + agg_ref[1]
    h = jnp.dot(a_ref[...], wt_ref[...],
                preferred_element_type=jnp.float32)
    h = h + jnp.dot(agg, wb_ref[...], preferred_element_type=jnp.float32)
    h = jax.nn.relu(h + bn_ref[...])
    gids = lax.broadcasted_iota(jnp.int32, (N_NODES, NUM_GRAPHS), 1)
    m = (n2g_ref[...] == gids).astype(jnp.float32)
    g_sum = lax.dot_general(m, h, (((0,), (0,)), ((), ())),
                            preferred_element_type=jnp.float32)
    ones = jnp.ones((N_NODES, 1), jnp.float32)
    g_cnt = lax.dot_general(m, ones, (((0,), (0,)), ((), ())),
                            preferred_element_type=jnp.float32)
    g_mean = g_sum / jnp.maximum(g_cnt, 1.0)
    h = h + jnp.dot(m, g_mean, preferred_element_type=jnp.float32)
    o_ref[...] = jax.nn.sigmoid(
        jnp.dot(h, wo_ref[...], preferred_element_type=jnp.float32)
        + bo_ref[...])


def _head(atom, agg2, n2g2d, w_top, w_bot, b_node2d, w_out, b_out2d):
    return pl.pallas_call(
        _head_body,
        out_shape=jax.ShapeDtypeStruct((N_NODES, 1), jnp.float32),
    )(atom, agg2, n2g2d, w_top, w_bot, b_node2d, w_out, b_out2d)


def kernel(atom_feature, edge_index, bond_feature, distance, b_factor,
           node2graph, W_msg, b_msg, W_node, b_node, W_out, b_out):
    src = edge_index[0]
    dst = edge_index[1]
    inv_scales = (1.0 / (2.0 ** jnp.arange(NUM_ENC, dtype=jnp.float32))
                  ).reshape(NUM_ENC, 1)

    d_raw = bond_feature.shape[1]
    p = _node_proj(atom_feature, W_msg[:D_NODE])
    q = _edge_q(bond_feature, distance.reshape(1, -1), inv_scales,
                W_msg[D_NODE:D_NODE + d_raw],
                W_msg[D_NODE + d_raw:D_NODE + d_raw + NUM_ENC],
                W_msg[D_NODE + d_raw + NUM_ENC:],
                b_msg.reshape(1, -1))
    agg2 = _sc_aggregate(p, q,
                         src.reshape(N_WORKERS, N_BATCH, IDXB, CHUNK),
                         dst.reshape(N_WORKERS, N_BATCH, IDXB, CHUNK))
    out = _head(atom_feature, agg2, node2graph.reshape(-1, 1),
                W_node[:D_NODE], W_node[D_NODE:], b_node.reshape(1, -1),
                W_out, b_out.reshape(1, 1))
    return out.reshape(-1), b_factor
